# baseline (speedup 1.0000x reference)
import numpy as np
import jax
import jax.numpy as jnp
from jax.experimental import pallas as pl
from jax.experimental.pallas import tpu as pltpu

N = 5000
B = 40

_NEG = float(np.finfo(np.float32).min)


def _toeplitz(wt, win, wout):
    kh, kw, ci, co = wt.shape
    sel = np.stack([np.eye(win, dtype=np.float32)[dx:dx + wout, :]
                    for dx in range(kw)])
    t = jnp.einsum('dox,edcf->excof', sel, wt)
    return t.reshape(kh, win * ci, wout * co)


def _prelu(y, a):
    return jnp.maximum(y, 0.0) + a * jnp.minimum(y, 0.0)


def _bdot(a, b):
    return jnp.dot(a.astype(jnp.bfloat16), b.astype(jnp.bfloat16),
                   preferred_element_type=jnp.float32)


def _onet_block(x_ref, w1_ref, w2_ref, w3_ref, w4_ref, w5_ref, w6_ref,
                b1_ref, a1_ref, b2_ref, a2_ref, b3_ref, a3_ref,
                b4_ref, a4_ref, b5_ref, a5_ref, b6_ref, out_ref):
    X = x_ref[...]

    y = jnp.broadcast_to(b1_ref[...], (B * 46, 1472))
    for dy in range(3):
        y = y + _bdot(X[:, dy:dy + 46, :].reshape(B * 46, 144), w1_ref[dy])
    y = _prelu(y, a1_ref[...]).reshape(B, 46, 23, 64)
    e, o = y[..., 0:32], y[..., 32:64]
    ep = jnp.concatenate([e, jnp.full((B, 46, 1, 32), _NEG, jnp.float32)],
                         axis=2)
    y = jnp.maximum(jnp.maximum(e, o), ep[:, :, 1:24])
    y = jnp.concatenate([y, jnp.full((B, 2, 23, 32), _NEG, jnp.float32)],
                        axis=1).reshape(B, 24, 2, 23, 32)
    e, o = y[:, :, 0], y[:, :, 1]
    y = jnp.maximum(jnp.maximum(e[:, 0:23], o[:, 0:23]), e[:, 1:24])
    p = y.reshape(B, 23, 736)

    y = jnp.broadcast_to(b2_ref[...], (B * 21, 1344))
    for dy in range(3):
        y = y + _bdot(p[:, dy:dy + 21, :].reshape(B * 21, 736), w2_ref[dy])
    y = _prelu(y, a2_ref[...]).reshape(B, 21, 1344)
    y = jnp.concatenate([y, jnp.full((B, 21, 64), _NEG, jnp.float32)],
                        axis=2).reshape(B, 21, 11, 128)
    e, o = y[..., 0:64], y[..., 64:128]
    y = jnp.maximum(jnp.maximum(e[:, :, 0:10], o[:, :, 0:10]), e[:, :, 1:11])
    y = jnp.concatenate([y, jnp.full((B, 1, 10, 64), _NEG, jnp.float32)],
                        axis=1).reshape(B, 11, 2, 10, 64)
    e, o = y[:, :, 0], y[:, :, 1]
    y = jnp.maximum(jnp.maximum(e[:, 0:10], o[:, 0:10]), e[:, 1:11])
    p = y.reshape(B, 10, 640)

    y = jnp.broadcast_to(b3_ref[...], (B * 8, 512))
    for dy in range(3):
        y = y + _bdot(p[:, dy:dy + 8, :].reshape(B * 8, 640), w3_ref[dy])
    y = _prelu(y, a3_ref[...]).reshape(B, 8, 4, 128)
    y = jnp.maximum(y[..., 0:64], y[..., 64:128])
    y = y.reshape(B, 4, 2, 4, 64)
    y = jnp.maximum(y[:, :, 0], y[:, :, 1])
    p = y.reshape(B, 4, 256)

    y = jnp.broadcast_to(b4_ref[...], (B * 3, 384))
    for dy in range(2):
        y = y + _bdot(p[:, dy:dy + 3, :].reshape(B * 3, 256), w4_ref[dy])
    y = _prelu(y, a4_ref[...]).reshape(B, 3, 384)

    y = _bdot(y.reshape(B, 1152), w5_ref[...]) + b5_ref[...]
    y = _prelu(y, a5_ref[...])
    z = _bdot(y, w6_ref[...]) + b6_ref[...]
    l = z[:, 14:16]
    m = jnp.max(l, axis=1, keepdims=True)
    e = jnp.exp(l - m)
    probs = e / jnp.sum(e, axis=1, keepdims=True)
    out_ref[...] = jnp.concatenate([z[:, 0:14], probs], axis=1)


def kernel(x, conv1_w, conv1_b, prelu1_a, conv2_w, conv2_b, prelu2_a,
           conv3_w, conv3_b, prelu3_a, conv4_w, conv4_b, prelu4_a,
           fc5_w, fc5_b, prelu5_a, fc61_w, fc61_b, fc62_w, fc62_b,
           fc63_w, fc63_b):
    n = x.shape[0]
    x3 = jnp.transpose(x, (0, 2, 3, 1)).reshape(n, 48, 144)

    w1 = _toeplitz(jnp.transpose(conv1_w, (2, 3, 1, 0)), 48, 46)
    w2 = _toeplitz(jnp.transpose(conv2_w, (2, 3, 1, 0)), 23, 21)
    w3 = _toeplitz(jnp.transpose(conv3_w, (2, 3, 1, 0)), 10, 8)
    w4 = _toeplitz(jnp.transpose(conv4_w, (2, 3, 1, 0)), 4, 3)
    w5 = jnp.transpose(fc5_w.reshape(256, 128, 3, 3), (3, 2, 1, 0)).reshape(1152, 256)
    w6 = jnp.concatenate([fc63_w, fc62_w, fc61_w], axis=0).T
    b6 = jnp.concatenate([fc63_b, fc62_b, fc61_b], axis=0)

    tile = lambda v, k: jnp.tile(v, k).reshape(1, -1)
    full = lambda a: pl.BlockSpec(a.shape, lambda i: (0,) * a.ndim)
    weights = [w1, w2, w3, w4, w5, w6,
               tile(conv1_b, 46), tile(prelu1_a, 46),
               tile(conv2_b, 21), tile(prelu2_a, 21),
               tile(conv3_b, 8), tile(prelu3_a, 8),
               tile(conv4_b, 3), tile(prelu4_a, 3),
               fc5_b.reshape(1, -1), prelu5_a.reshape(1, -1),
               b6.reshape(1, -1)]

    out = pl.pallas_call(
        _onet_block,
        grid=(n // B,),
        in_specs=[pl.BlockSpec((B, 48, 144), lambda i: (i, 0, 0))]
                 + [full(a) for a in weights],
        out_specs=pl.BlockSpec((B, 16), lambda i: (i, 0)),
        out_shape=jax.ShapeDtypeStruct((n, 16), jnp.float32),
        compiler_params=pltpu.CompilerParams(
            dimension_semantics=("parallel",)),
    )(x3, *weights)

    return out[:, 0:10], out[:, 10:14], out[:, 14:16]


# speedup vs baseline: 1.1401x; 1.1401x over previous
import numpy as np
import jax
import jax.numpy as jnp
from jax.experimental import pallas as pl
from jax.experimental.pallas import tpu as pltpu

N = 5000
B = 40

_NEG = float(np.finfo(np.float32).min)
_BF = jnp.bfloat16


def _toeplitz(wt, win, wout):
    kh, kw, ci, co = wt.shape
    sel = np.stack([np.eye(win, dtype=np.float32)[dx:dx + wout, :]
                    for dx in range(kw)])
    t = jnp.einsum('dox,edcf->excof', sel, wt)
    return t.reshape(kh, win * ci, wout * co)


def _act(acc, b, a):
    y = (acc + b).astype(_BF)
    return jnp.where(y >= 0, y, a * y)


def _dot(a, b):
    return jnp.dot(a, b, preferred_element_type=jnp.float32)


def _onet_block(x_ref, w1_ref, w2_ref, w3_ref, w4_ref, w5_ref, w6_ref,
                b1_ref, a1_ref, b2_ref, a2_ref, b3_ref, a3_ref,
                b4_ref, a4_ref, b5_ref, a5_ref, b6_ref, out_ref):
    X = x_ref[...].astype(_BF)

    acc = jnp.broadcast_to(jnp.float32(0.0), (B * 46, 1472))
    for dy in range(3):
        acc = acc + _dot(X[:, dy:dy + 46, :].reshape(B * 46, 144), w1_ref[dy])
    y = _act(acc, b1_ref[...], a1_ref[...]).reshape(B, 46, 23, 64)
    e, o = y[..., 0:32], y[..., 32:64]
    ep = jnp.concatenate([e, jnp.full((B, 46, 1, 32), _NEG, _BF)], axis=2)
    y = jnp.maximum(jnp.maximum(e, o), ep[:, :, 1:24])
    y = jnp.concatenate([y, jnp.full((B, 2, 23, 32), _NEG, _BF)],
                        axis=1).reshape(B, 24, 2, 23, 32)
    e, o = y[:, :, 0], y[:, :, 1]
    y = jnp.maximum(jnp.maximum(e[:, 0:23], o[:, 0:23]), e[:, 1:24])
    p = y.reshape(B, 23, 736)

    acc = jnp.broadcast_to(jnp.float32(0.0), (B * 21, 1344))
    for dy in range(3):
        acc = acc + _dot(p[:, dy:dy + 21, :].reshape(B * 21, 736), w2_ref[dy])
    y = _act(acc, b2_ref[...], a2_ref[...]).reshape(B, 21, 1344)
    y = jnp.concatenate([y, jnp.full((B, 21, 64), _NEG, _BF)],
                        axis=2).reshape(B, 21, 11, 128)
    e, o = y[..., 0:64], y[..., 64:128]
    y = jnp.maximum(jnp.maximum(e[:, :, 0:10], o[:, :, 0:10]), e[:, :, 1:11])
    y = jnp.concatenate([y, jnp.full((B, 1, 10, 64), _NEG, _BF)],
                        axis=1).reshape(B, 11, 2, 10, 64)
    e, o = y[:, :, 0], y[:, :, 1]
    y = jnp.maximum(jnp.maximum(e[:, 0:10], o[:, 0:10]), e[:, 1:11])
    p = y.reshape(B, 10, 640)

    acc = jnp.broadcast_to(jnp.float32(0.0), (B * 8, 512))
    for dy in range(3):
        acc = acc + _dot(p[:, dy:dy + 8, :].reshape(B * 8, 640), w3_ref[dy])
    y = _act(acc, b3_ref[...], a3_ref[...]).reshape(B, 8, 4, 128)
    y = jnp.maximum(y[..., 0:64], y[..., 64:128])
    y = y.reshape(B, 4, 2, 4, 64)
    y = jnp.maximum(y[:, :, 0], y[:, :, 1])
    p = y.reshape(B, 4, 256)

    acc = jnp.broadcast_to(jnp.float32(0.0), (B * 3, 384))
    for dy in range(2):
        acc = acc + _dot(p[:, dy:dy + 3, :].reshape(B * 3, 256), w4_ref[dy])
    y = _act(acc, b4_ref[...], a4_ref[...]).reshape(B, 3, 384)

    y = _act(_dot(y.reshape(B, 1152), w5_ref[...]),
             b5_ref[...], a5_ref[...])
    z = _dot(y, w6_ref[...]) + b6_ref[...]
    l = z[:, 14:16]
    m = jnp.max(l, axis=1, keepdims=True)
    e = jnp.exp(l - m)
    probs = e / jnp.sum(e, axis=1, keepdims=True)
    out_ref[...] = jnp.concatenate([z[:, 0:14], probs], axis=1)


def kernel(x, conv1_w, conv1_b, prelu1_a, conv2_w, conv2_b, prelu2_a,
           conv3_w, conv3_b, prelu3_a, conv4_w, conv4_b, prelu4_a,
           fc5_w, fc5_b, prelu5_a, fc61_w, fc61_b, fc62_w, fc62_b,
           fc63_w, fc63_b):
    n = x.shape[0]
    x3 = jnp.transpose(x, (0, 2, 3, 1)).reshape(n, 48, 144)

    bf = lambda a: a.astype(_BF)
    w1 = bf(_toeplitz(jnp.transpose(conv1_w, (2, 3, 1, 0)), 48, 46))
    w2 = bf(_toeplitz(jnp.transpose(conv2_w, (2, 3, 1, 0)), 23, 21))
    w3 = bf(_toeplitz(jnp.transpose(conv3_w, (2, 3, 1, 0)), 10, 8))
    w4 = bf(_toeplitz(jnp.transpose(conv4_w, (2, 3, 1, 0)), 4, 3))
    w5 = bf(jnp.transpose(fc5_w.reshape(256, 128, 3, 3), (3, 2, 1, 0)).reshape(1152, 256))
    w6 = bf(jnp.concatenate([fc63_w, fc62_w, fc61_w], axis=0).T)
    b6 = jnp.concatenate([fc63_b, fc62_b, fc61_b], axis=0)

    tile = lambda v, k: jnp.tile(v, k).reshape(1, -1)
    btile = lambda v, k: bf(jnp.tile(v, k).reshape(1, -1))
    full = lambda a: pl.BlockSpec(a.shape, lambda i: (0,) * a.ndim)
    weights = [w1, w2, w3, w4, w5, w6,
               tile(conv1_b, 46), btile(prelu1_a, 46),
               tile(conv2_b, 21), btile(prelu2_a, 21),
               tile(conv3_b, 8), btile(prelu3_a, 8),
               tile(conv4_b, 3), btile(prelu4_a, 3),
               fc5_b.reshape(1, -1), bf(prelu5_a.reshape(1, -1)),
               b6.reshape(1, -1)]

    out = pl.pallas_call(
        _onet_block,
        grid=(n // B,),
        in_specs=[pl.BlockSpec((B, 48, 144), lambda i: (i, 0, 0))]
                 + [full(a) for a in weights],
        out_specs=pl.BlockSpec((B, 16), lambda i: (i, 0)),
        out_shape=jax.ShapeDtypeStruct((n, 16), jnp.float32),
        compiler_params=pltpu.CompilerParams(
            dimension_semantics=("parallel",)),
    )(x3, *weights)

    return out[:, 0:10], out[:, 10:14], out[:, 14:16]
